# baseline (speedup 1.0000x reference)
import jax
import jax.numpy as jnp
from jax import lax
from jax.experimental import pallas as pl
from jax.experimental.pallas import tpu as pltpu
from jax.experimental.pallas import tpu_sc as plsc

RADIUS2 = 0.2 ** 2
N_PTS = 16384
N_CTR = 1024
N_BATCH = 4
K_OUT = 32
NUM_CORES = 2
NUM_SUBCORES = 16
NW = NUM_CORES * NUM_SUBCORES
ROWS_PER_W = (N_BATCH * N_CTR) // NW
W_PER_BATCH = NW // N_BATCH
CHUNKS = N_PTS // 16
UNROLL = 16
CLAMP = 64


STAGE = 512


MASKHI = -65536


def _sc_body(pos_hbm, posb_hbm, pw_hbm, cent_hbm, out_hbm,
             tx_v, ty_v, tz_v, pw_v, pzb_v, p2_v, c2_v,
             cent_v, row_v, out_v):
    wid = lax.axis_index("s") * NUM_CORES + lax.axis_index("c")
    b = wid // W_PER_BATCH
    pbase = b * 3 * N_PTS
    pltpu.sync_copy(pw_hbm.at[pl.ds(b * N_PTS, N_PTS)], pw_v)
    pltpu.sync_copy(posb_hbm.at[pl.ds(pbase + 2 * N_PTS, N_PTS)], pzb_v)
    pltpu.sync_copy(cent_hbm.at[pl.ds(wid * ROWS_PER_W, ROWS_PER_W)], cent_v)

    iota = lax.iota(jnp.int32, 16)
    zeros16 = jnp.zeros((16,), jnp.int32)

    def stage_blk(t, carry):
        base = t * STAGE
        pltpu.sync_copy(pos_hbm.at[pl.ds(pbase + base, STAGE)], tx_v)
        pltpu.sync_copy(pos_hbm.at[pl.ds(pbase + N_PTS + base, STAGE)], ty_v)
        pltpu.sync_copy(pos_hbm.at[pl.ds(pbase + 2 * N_PTS + base, STAGE)], tz_v)

        def norm_body(k, c):
            px = tx_v[pl.ds(k * 16, 16)]
            py = ty_v[pl.ds(k * 16, 16)]
            pz = tz_v[pl.ds(k * 16, 16)]
            p2_v[pl.ds(base + k * 16, 16)] = (px * px + py * py) + pz * pz
            c2_v[pl.ds(base + k * 16, 16)] = (px * px + pz * pz) + py * py
            return c

        lax.fori_loop(0, STAGE // 16, norm_body, 0)
        return carry

    lax.fori_loop(0, N_PTS // STAGE, stage_blk, 0)

    ones16 = jnp.ones((16,), jnp.int32)

    def load_row(r):
        rsplat = jnp.full((16,), r, jnp.int32)
        cidx = plsc.load_gather(cent_v, [rsplat])
        cw = plsc.load_gather(pw_v, [cidx])
        cx2 = -2.0 * plsc.bitcast(cw & MASKHI, jnp.float32)
        cy2 = -2.0 * plsc.bitcast(cw << 16, jnp.float32)
        cz2 = -2.0 * plsc.load_gather(pzb_v, [cidx])
        c2 = plsc.load_gather(c2_v, [cidx])
        return cx2, cy2, cz2, c2

    def emit_row(r, cnt, roff):
        firstv = plsc.load_gather(row_v, [jnp.full((16,), roff, jnp.int32)])
        firstv = jnp.where(cnt > 0, firstv, jnp.full((16,), N_PTS, jnp.int32))
        cnt_splat = jnp.full((16,), cnt, jnp.int32)
        o0 = jnp.where(iota < cnt_splat, row_v[pl.ds(roff, 16)], firstv)
        o1 = jnp.where(iota + 16 < cnt_splat, row_v[pl.ds(roff + 16, 16)], firstv)
        out_v[pl.ds(r * K_OUT, 16)] = o0
        out_v[pl.ds(r * K_OUT + 16, 16)] = o1

    def row_body(r, carry):
        cx2, cy2, cz2, c2 = load_row(r)

        def compute_block(j):
            outs = []
            for u in range(UNROLL):
                base = (j + u) * 16
                w = pw_v[pl.ds(base, 16)]
                pxb = plsc.bitcast(w & MASKHI, jnp.float32)
                pyb = plsc.bitcast(w << 16, jnp.float32)
                pzb = pzb_v[pl.ds(base, 16)]
                p2 = p2_v[pl.ds(base, 16)]
                d = (((cx2 * pxb + cy2 * pyb) + cz2 * pzb) + c2) + p2
                pcs = plsc.all_reduce_population_count(d <= RADIUS2)[0]
                outs.append((d, pcs))
            return tuple(x for o in outs for x in o)

        def append_block(j, cnt, pend):
            for u in range(UNROLL):
                dvec, pcs = pend[2 * u], pend[2 * u + 1]
                mask = dvec <= RADIUS2
                idxv = jnp.full((16,), (j - UNROLL + u) * 16, jnp.int32) + iota
                plsc.store_compressed(
                    row_v.at[pl.ds(jnp.minimum(cnt, CLAMP), 16)], idxv, mask=mask)
                cnt = cnt + pcs
            return cnt

        def cond(state):
            j, cnt = state[0], state[1]
            return jnp.logical_and(j < CHUNKS, cnt < K_OUT)

        def block(state):
            j, cnt = state[0], state[1]
            pend = state[2:]
            cnt = append_block(j, cnt, pend)
            npend = compute_block(j)
            return (j + UNROLL, cnt) + npend

        big16 = jnp.full((16,), 1.0e9, jnp.float32)
        init = (jnp.int32(0), jnp.int32(0)) + \
            tuple(x for _ in range(UNROLL) for x in (big16, jnp.int32(0)))
        fin = lax.while_loop(cond, block, init)
        cnt = append_block(fin[0], fin[1], fin[2:])
        emit_row(r, cnt, 0)
        return carry

    lax.fori_loop(0, ROWS_PER_W, row_body, 0)
    pltpu.sync_copy(
        out_v, out_hbm.at[pl.ds(wid * ROWS_PER_W * K_OUT, ROWS_PER_W * K_OUT)])


def _make_call():
    return pl.kernel(
        _sc_body,
        out_type=jax.ShapeDtypeStruct((N_BATCH * N_CTR * K_OUT,), jnp.int32),
        name="frnn_sc",
        compiler_params=pltpu.CompilerParams(needs_layout_passes=False),
        mesh=plsc.VectorSubcoreMesh(
            core_axis_name="c", subcore_axis_name="s",
            num_cores=NUM_CORES, num_subcores=NUM_SUBCORES),
        scratch_types=[
            pltpu.VMEM((STAGE,), jnp.float32),
            pltpu.VMEM((STAGE,), jnp.float32),
            pltpu.VMEM((STAGE,), jnp.float32),
            pltpu.VMEM((N_PTS,), jnp.int32),
            pltpu.VMEM((N_PTS,), jnp.float32),
            pltpu.VMEM((N_PTS,), jnp.float32),
            pltpu.VMEM((N_PTS,), jnp.float32),
            pltpu.VMEM((ROWS_PER_W,), jnp.int32),
            pltpu.VMEM((256,), jnp.int32),
            pltpu.VMEM((ROWS_PER_W * K_OUT,), jnp.int32),
        ],
    )


def _round_bf16(x):
    u = jax.lax.bitcast_convert_type(x, jnp.uint32)
    u = (u + jnp.uint32(0x7FFF) + ((u >> jnp.uint32(16)) & jnp.uint32(1)))
    u = u & jnp.uint32(0xFFFF0000)
    return jax.lax.bitcast_convert_type(u, jnp.float32)


def kernel(pos, centroids):
    pos_t = jnp.transpose(pos, (0, 2, 1)).reshape(-1)
    pos_b = _round_bf16(pos_t)
    pb3 = pos_b.reshape(N_BATCH, 3, N_PTS)
    xbits = jax.lax.bitcast_convert_type(pb3[:, 0], jnp.uint32)
    ybits = jax.lax.bitcast_convert_type(pb3[:, 1], jnp.uint32)
    pw = jax.lax.bitcast_convert_type(
        xbits | (ybits >> jnp.uint32(16)), jnp.int32).reshape(-1)
    cent = centroids.reshape(-1).astype(jnp.int32)
    out = _make_call()(pos_t, pos_b, pw, cent)
    return out.reshape(N_BATCH, N_CTR, K_OUT)


# speedup vs baseline: 1.4226x; 1.4226x over previous
import jax
import jax.numpy as jnp
from jax import lax
from jax.experimental import pallas as pl
from jax.experimental.pallas import tpu as pltpu
from jax.experimental.pallas import tpu_sc as plsc

RADIUS2 = 0.2 ** 2
N_PTS = 16384
N_CTR = 1024
N_BATCH = 4
K_OUT = 32
NUM_CORES = 2
NUM_SUBCORES = 16
NW = NUM_CORES * NUM_SUBCORES
ROWS_PER_W = (N_BATCH * N_CTR) // NW
W_PER_BATCH = NW // N_BATCH
CHUNKS = N_PTS // 16
UNROLL = 16
CLAMP = 64


STAGE = 512


MASKHI = -65536


def _sc_body(pos_hbm, posb_hbm, pw_hbm, cent_hbm, out_hbm,
             tx_v, ty_v, tz_v, pw_v, pzb_v, p2_v, c2_v,
             cent_v, row_v, out_v):
    wid = lax.axis_index("s") * NUM_CORES + lax.axis_index("c")
    b = wid // W_PER_BATCH
    pbase = b * 3 * N_PTS
    pltpu.sync_copy(pw_hbm.at[pl.ds(b * N_PTS, N_PTS)], pw_v)
    pltpu.sync_copy(posb_hbm.at[pl.ds(pbase + 2 * N_PTS, N_PTS)], pzb_v)
    pltpu.sync_copy(cent_hbm.at[pl.ds(wid * ROWS_PER_W, ROWS_PER_W)], cent_v)

    iota = lax.iota(jnp.int32, 16)
    zeros16 = jnp.zeros((16,), jnp.int32)

    def stage_blk(t, carry):
        base = t * STAGE
        pltpu.sync_copy(pos_hbm.at[pl.ds(pbase + base, STAGE)], tx_v)
        pltpu.sync_copy(pos_hbm.at[pl.ds(pbase + N_PTS + base, STAGE)], ty_v)
        pltpu.sync_copy(pos_hbm.at[pl.ds(pbase + 2 * N_PTS + base, STAGE)], tz_v)

        def norm_body(k, c):
            px = tx_v[pl.ds(k * 16, 16)]
            py = ty_v[pl.ds(k * 16, 16)]
            pz = tz_v[pl.ds(k * 16, 16)]
            p2_v[pl.ds(base + k * 16, 16)] = (px * px + py * py) + pz * pz
            c2_v[pl.ds(base + k * 16, 16)] = (px * px + pz * pz) + py * py
            return c

        lax.fori_loop(0, STAGE // 16, norm_body, 0)
        return carry

    lax.fori_loop(0, N_PTS // STAGE, stage_blk, 0)

    ones16 = jnp.ones((16,), jnp.int32)

    def load_row(r):
        rsplat = jnp.full((16,), r, jnp.int32)
        cidx = plsc.load_gather(cent_v, [rsplat])
        cw = plsc.load_gather(pw_v, [cidx])
        cx2 = -2.0 * plsc.bitcast(cw & MASKHI, jnp.float32)
        cy2 = -2.0 * plsc.bitcast(cw << 16, jnp.float32)
        cz2 = -2.0 * plsc.load_gather(pzb_v, [cidx])
        c2 = plsc.load_gather(c2_v, [cidx])
        return cx2, cy2, cz2, c2

    def emit_row(r, cnt, roff):
        firstv = plsc.load_gather(row_v, [jnp.full((16,), roff, jnp.int32)])
        firstv = jnp.where(cnt > 0, firstv, jnp.full((16,), N_PTS, jnp.int32))
        cnt_splat = jnp.full((16,), cnt, jnp.int32)
        o0 = jnp.where(iota < cnt_splat, row_v[pl.ds(roff, 16)], firstv)
        o1 = jnp.where(iota + 16 < cnt_splat, row_v[pl.ds(roff + 16, 16)], firstv)
        out_v[pl.ds(r * K_OUT, 16)] = o0
        out_v[pl.ds(r * K_OUT + 16, 16)] = o1

    def row_body(r, carry):
        cx2, cy2, cz2, c2 = load_row(r)

        def cond(state):
            j, cnt = state
            return jnp.logical_and(j < CHUNKS, cnt < K_OUT)

        def block(state):
            j, cnt = state
            masks = []
            for u in range(UNROLL):
                base = (j + u) * 16
                w = pw_v[pl.ds(base, 16)]
                pxb = plsc.bitcast(w & MASKHI, jnp.float32)
                pyb = plsc.bitcast(w << 16, jnp.float32)
                pzb = pzb_v[pl.ds(base, 16)]
                p2 = p2_v[pl.ds(base, 16)]
                d = (((cx2 * pxb + cy2 * pyb) + cz2 * pzb) + c2) + p2
                mask = d <= RADIUS2
                pc = plsc.all_reduce_population_count(mask)
                masks.append((base, mask, pc))
            for base, mask, pc in masks:
                idxv = jnp.full((16,), base, jnp.int32) + iota
                plsc.store_compressed(row_v.at[pl.ds(cnt, 16)], idxv, mask=mask)
                cnt = cnt + pc[0]
            return j + UNROLL, cnt

        _, cnt = lax.while_loop(cond, block, (jnp.int32(0), jnp.int32(0)))
        emit_row(r, cnt, 0)
        return carry

    lax.fori_loop(0, ROWS_PER_W, row_body, 0)
    pltpu.sync_copy(
        out_v, out_hbm.at[pl.ds(wid * ROWS_PER_W * K_OUT, ROWS_PER_W * K_OUT)])


def _make_call():
    return pl.kernel(
        _sc_body,
        out_type=jax.ShapeDtypeStruct((N_BATCH * N_CTR * K_OUT,), jnp.int32),
        name="frnn_sc",
        compiler_params=pltpu.CompilerParams(needs_layout_passes=False),
        mesh=plsc.VectorSubcoreMesh(
            core_axis_name="c", subcore_axis_name="s",
            num_cores=NUM_CORES, num_subcores=NUM_SUBCORES),
        scratch_types=[
            pltpu.VMEM((STAGE,), jnp.float32),
            pltpu.VMEM((STAGE,), jnp.float32),
            pltpu.VMEM((STAGE,), jnp.float32),
            pltpu.VMEM((N_PTS,), jnp.int32),
            pltpu.VMEM((N_PTS,), jnp.float32),
            pltpu.VMEM((N_PTS,), jnp.float32),
            pltpu.VMEM((N_PTS,), jnp.float32),
            pltpu.VMEM((ROWS_PER_W,), jnp.int32),
            pltpu.VMEM((32 + 3 * 16 * UNROLL + 16,), jnp.int32),
            pltpu.VMEM((ROWS_PER_W * K_OUT,), jnp.int32),
        ],
    )


def _round_bf16(x):
    u = jax.lax.bitcast_convert_type(x, jnp.uint32)
    u = (u + jnp.uint32(0x7FFF) + ((u >> jnp.uint32(16)) & jnp.uint32(1)))
    u = u & jnp.uint32(0xFFFF0000)
    return jax.lax.bitcast_convert_type(u, jnp.float32)


def kernel(pos, centroids):
    pos_t = jnp.transpose(pos, (0, 2, 1)).reshape(-1)
    pos_b = _round_bf16(pos_t)
    pb3 = pos_b.reshape(N_BATCH, 3, N_PTS)
    xbits = jax.lax.bitcast_convert_type(pb3[:, 0], jnp.uint32)
    ybits = jax.lax.bitcast_convert_type(pb3[:, 1], jnp.uint32)
    pw = jax.lax.bitcast_convert_type(
        xbits | (ybits >> jnp.uint32(16)), jnp.int32).reshape(-1)
    cent = centroids.reshape(-1).astype(jnp.int32)
    out = _make_call()(pos_t, pos_b, pw, cent)
    return out.reshape(N_BATCH, N_CTR, K_OUT)


# speedup vs baseline: 1.4240x; 1.0010x over previous
import jax
import jax.numpy as jnp
from jax import lax
from jax.experimental import pallas as pl
from jax.experimental.pallas import tpu as pltpu
from jax.experimental.pallas import tpu_sc as plsc

RADIUS2 = 0.2 ** 2
N_PTS = 16384
N_CTR = 1024
N_BATCH = 4
K_OUT = 32
NUM_CORES = 2
NUM_SUBCORES = 16
NW = NUM_CORES * NUM_SUBCORES
ROWS_PER_W = (N_BATCH * N_CTR) // NW
W_PER_BATCH = NW // N_BATCH
CHUNKS = N_PTS // 16
UNROLL = 16


STAGE = 512


MASKHI = -65536


def _sc_body(pos_hbm, posb_hbm, pw_hbm, cent_hbm, out_hbm,
             tx_v, ty_v, tz_v, pw_v, pzb_v, p2_v, c2_v,
             cent_v, row_v, out_v):
    wid = lax.axis_index("s") * NUM_CORES + lax.axis_index("c")
    b = wid // W_PER_BATCH
    pbase = b * 3 * N_PTS
    pltpu.sync_copy(pw_hbm.at[pl.ds(b * N_PTS, N_PTS)], pw_v)
    pltpu.sync_copy(posb_hbm.at[pl.ds(pbase + 2 * N_PTS, N_PTS)], pzb_v)
    pltpu.sync_copy(cent_hbm.at[pl.ds(wid * ROWS_PER_W, ROWS_PER_W)], cent_v)

    iota = lax.iota(jnp.int32, 16)

    def stage_blk(t, carry):
        base = t * STAGE
        pltpu.sync_copy(pos_hbm.at[pl.ds(pbase + base, STAGE)], tx_v)
        pltpu.sync_copy(pos_hbm.at[pl.ds(pbase + N_PTS + base, STAGE)], ty_v)
        pltpu.sync_copy(pos_hbm.at[pl.ds(pbase + 2 * N_PTS + base, STAGE)], tz_v)

        def norm_body(k, c):
            px = tx_v[pl.ds(k * 16, 16)]
            py = ty_v[pl.ds(k * 16, 16)]
            pz = tz_v[pl.ds(k * 16, 16)]
            p2_v[pl.ds(base + k * 16, 16)] = (px * px + py * py) + pz * pz
            c2_v[pl.ds(base + k * 16, 16)] = (px * px + pz * pz) + py * py
            return c

        lax.fori_loop(0, STAGE // 16, norm_body, 0)
        return carry

    lax.fori_loop(0, N_PTS // STAGE, stage_blk, 0)

    def load_row(r):
        rsplat = jnp.full((16,), r, jnp.int32)
        cidx = plsc.load_gather(cent_v, [rsplat])
        cw = plsc.load_gather(pw_v, [cidx])
        cx2 = -2.0 * plsc.bitcast(cw & MASKHI, jnp.float32)
        cy2 = -2.0 * plsc.bitcast(cw << 16, jnp.float32)
        cz2 = -2.0 * plsc.load_gather(pzb_v, [cidx])
        c2 = plsc.load_gather(c2_v, [cidx])
        return cx2, cy2, cz2, c2

    def emit_row(r, cnt, roff):
        firstv = plsc.load_gather(row_v, [jnp.full((16,), roff, jnp.int32)])
        firstv = jnp.where(cnt > 0, firstv, jnp.full((16,), N_PTS, jnp.int32))
        cnt_splat = jnp.full((16,), cnt, jnp.int32)
        o0 = jnp.where(iota < cnt_splat, row_v[pl.ds(roff, 16)], firstv)
        o1 = jnp.where(iota + 16 < cnt_splat, row_v[pl.ds(roff + 16, 16)], firstv)
        out_v[pl.ds(r * K_OUT, 16)] = o0
        out_v[pl.ds(r * K_OUT + 16, 16)] = o1

    def row_body(r, carry):
        cx2, cy2, cz2, c2 = load_row(r)

        def cond(state):
            j, cnt = state
            return jnp.logical_and(j < CHUNKS, cnt < K_OUT)

        def block(state):
            j, cnt = state
            masks = []
            for u in range(UNROLL):
                base = (j + u) * 16
                w = pw_v[pl.ds(base, 16)]
                pxb = plsc.bitcast(w & MASKHI, jnp.float32)
                pyb = plsc.bitcast(w << 16, jnp.float32)
                pzb = pzb_v[pl.ds(base, 16)]
                p2 = p2_v[pl.ds(base, 16)]
                d = (((cx2 * pxb + cy2 * pyb) + cz2 * pzb) + c2) + p2
                mask = d <= RADIUS2
                pc = plsc.all_reduce_population_count(mask)
                masks.append((base, mask, pc))
            for base, mask, pc in masks:
                idxv = jnp.full((16,), base, jnp.int32) + iota
                plsc.store_compressed(row_v.at[pl.ds(cnt, 16)], idxv, mask=mask)
                cnt = cnt + pc[0]
            return j + UNROLL, cnt

        _, cnt = lax.while_loop(cond, block, (jnp.int32(0), jnp.int32(0)))
        emit_row(r, cnt, 0)
        return carry

    lax.fori_loop(0, ROWS_PER_W, row_body, 0)
    pltpu.sync_copy(
        out_v, out_hbm.at[pl.ds(wid * ROWS_PER_W * K_OUT, ROWS_PER_W * K_OUT)])


def _make_call():
    return pl.kernel(
        _sc_body,
        out_type=jax.ShapeDtypeStruct((N_BATCH * N_CTR * K_OUT,), jnp.int32),
        name="frnn_sc",
        compiler_params=pltpu.CompilerParams(needs_layout_passes=False),
        mesh=plsc.VectorSubcoreMesh(
            core_axis_name="c", subcore_axis_name="s",
            num_cores=NUM_CORES, num_subcores=NUM_SUBCORES),
        scratch_types=[
            pltpu.VMEM((STAGE,), jnp.float32),
            pltpu.VMEM((STAGE,), jnp.float32),
            pltpu.VMEM((STAGE,), jnp.float32),
            pltpu.VMEM((N_PTS,), jnp.int32),
            pltpu.VMEM((N_PTS,), jnp.float32),
            pltpu.VMEM((N_PTS,), jnp.float32),
            pltpu.VMEM((N_PTS,), jnp.float32),
            pltpu.VMEM((ROWS_PER_W,), jnp.int32),
            pltpu.VMEM((32 + 3 * 16 * UNROLL + 16,), jnp.int32),
            pltpu.VMEM((ROWS_PER_W * K_OUT,), jnp.int32),
        ],
    )


def _round_bf16(x):
    u = jax.lax.bitcast_convert_type(x, jnp.uint32)
    u = (u + jnp.uint32(0x7FFF) + ((u >> jnp.uint32(16)) & jnp.uint32(1)))
    u = u & jnp.uint32(0xFFFF0000)
    return jax.lax.bitcast_convert_type(u, jnp.float32)


def kernel(pos, centroids):
    pos_t = jnp.transpose(pos, (0, 2, 1)).reshape(-1)
    pos_b = _round_bf16(pos_t)
    pb3 = pos_b.reshape(N_BATCH, 3, N_PTS)
    xbits = jax.lax.bitcast_convert_type(pb3[:, 0], jnp.uint32)
    ybits = jax.lax.bitcast_convert_type(pb3[:, 1], jnp.uint32)
    pw = jax.lax.bitcast_convert_type(
        xbits | (ybits >> jnp.uint32(16)), jnp.int32).reshape(-1)
    cent = centroids.reshape(-1).astype(jnp.int32)
    out = _make_call()(pos_t, pos_b, pw, cent)
    return out.reshape(N_BATCH, N_CTR, K_OUT)
